# Initial kernel scaffold; baseline (speedup 1.0000x reference)
import jax
import jax.numpy as jnp
from jax.experimental import pallas as pl


def kernel(features, edge_index, edge_features, W_neigh, b_neigh, W_edge, b_edge, W_lin, b_lin):
    raise NotImplementedError("write your pallas kernel here")


# speedup vs baseline: 2.1423x; 2.1423x over previous
import functools

import jax
import jax.numpy as jnp
from jax import lax
from jax.experimental import pallas as pl
from jax.experimental.pallas import tpu as pltpu
from jax.experimental.pallas import tpu_sc as plsc

_CHUNK = 64
_NSUB = 16
_NTILE = 32


def _sc_gather_segsum(f2, src2, dstg, z_acc, n_pad, half, chg, e_pad):
  rows_per_tile = n_pad // _NSUB
  mesh = plsc.VectorSubcoreMesh(core_axis_name="c", subcore_axis_name="s")

  @functools.partial(
      pl.kernel,
      mesh=mesh,
      out_type=jax.ShapeDtypeStruct((2, n_pad, half), jnp.float32),
      scratch_types=[
          pltpu.VMEM_SHARED((n_pad, half), jnp.float32),
          pltpu.VMEM((_CHUNK,), jnp.int32),
          pltpu.VMEM((_CHUNK,), jnp.int32),
          pltpu.VMEM((_CHUNK, half), jnp.float32),
          pltpu.SemaphoreType.DMA,
      ],
  )
  def k(f2_h, src2_h, dstg_h, zacc_h, g2_h, acc_sh, src_v, dst_v, rows_v, sem):
    cid = lax.axis_index("c")
    sid = lax.axis_index("s")
    r0 = sid * rows_per_tile

    pltpu.sync_copy(zacc_h.at[pl.ds(r0, rows_per_tile)],
                    acc_sh.at[pl.ds(r0, rows_per_tile)])
    plsc.subcore_barrier()

    gbase = sid * (chg * _CHUNK)

    def g_body(i, carry):
      pltpu.sync_copy(src2_h.at[pl.ds(cid * e_pad + gbase + i * _CHUNK,
                                      _CHUNK)], src_v)
      pltpu.sync_copy(dstg_h.at[pl.ds(gbase + i * _CHUNK, _CHUNK)], dst_v)
      pltpu.async_copy(f2_h.at[src_v], rows_v, sem).wait()
      pltpu.sync_copy(rows_v, acc_sh.at[dst_v], add=True)
      return carry

    lax.fori_loop(0, chg, g_body, 0)
    plsc.subcore_barrier()

    pltpu.sync_copy(acc_sh.at[pl.ds(r0, rows_per_tile)],
                    g2_h.at[cid, pl.ds(r0, rows_per_tile)])

  return k(f2, src2, dstg, z_acc)


def _sc_edge_segsum(ef2, dstg, z_fe, n_pad, chf, e_pad):
  rows_per_tile = n_pad // _NSUB
  mesh = plsc.VectorSubcoreMesh(core_axis_name="c", subcore_axis_name="s")

  @functools.partial(
      pl.kernel,
      mesh=mesh,
      out_type=jax.ShapeDtypeStruct((2, n_pad, 128), jnp.float32),
      scratch_types=[
          pltpu.VMEM_SHARED((n_pad, 128), jnp.float32),
          pltpu.VMEM((_CHUNK,), jnp.int32),
          pltpu.VMEM((_CHUNK, 128), jnp.float32),
      ],
  )
  def k(ef2_h, dstg_h, zfe_h, fe2_h, fe_sh, dst_v, ef_v):
    cid = lax.axis_index("c")
    sid = lax.axis_index("s")
    r0 = sid * rows_per_tile

    pltpu.sync_copy(zfe_h.at[pl.ds(r0, rows_per_tile)],
                    fe_sh.at[pl.ds(r0, rows_per_tile)])
    plsc.subcore_barrier()

    fbase = (cid * _NSUB + sid) * (chf * _CHUNK)

    def f_body(i, carry):
      off = fbase + i * _CHUNK
      pltpu.sync_copy(ef2_h.at[pl.ds(off, _CHUNK)], ef_v)
      pltpu.sync_copy(dstg_h.at[pl.ds(off, _CHUNK)], dst_v)
      pltpu.sync_copy(ef_v, fe_sh.at[dst_v], add=True)
      return carry

    lax.fori_loop(0, chf, f_body, 0)
    plsc.subcore_barrier()

    pltpu.sync_copy(fe_sh.at[pl.ds(r0, rows_per_tile)],
                    fe2_h.at[cid, pl.ds(r0, rows_per_tile)])

  return k(ef2, dstg, z_fe)


def _tc_epilogue(features, gl, gr, fea, feb, W_neigh, W_edge, W_lin,
                 b_neigh, b_edge, b_lin, n, d, half, de):
  bm = 1000
  grid = (n // bm,)

  def body(f_ref, gl_ref, gr_ref, fea_ref, feb_ref, wn_ref, we_ref, wl_ref,
           bn_ref, be_ref, bl_ref, out_ref):
    dot = lambda a, b: lax.dot_general(
        a, b, (((1,), (1,)), ((), ())), preferred_element_type=jnp.float32)
    f = f_ref[...]
    fe = fea_ref[...] + feb_ref[...]
    counts = fe[:, de:de + 1]
    hsum = dot(gl_ref[...], wn_ref[:, :half]) + dot(gr_ref[...], wn_ref[:, half:])
    sums = hsum + dot(fe[:, :de], we_ref[...])
    sums = sums + counts * (bn_ref[...] + be_ref[...]) + f
    s = sums / jnp.maximum(counts, 1.0)
    out_ref[...] = dot(f, wl_ref[:, :d]) + dot(s, wl_ref[:, d:]) + bl_ref[...]

  de2 = fea.shape[1]
  return pl.pallas_call(
      body,
      grid=grid,
      in_specs=[
          pl.BlockSpec((bm, d), lambda i: (i, 0)),
          pl.BlockSpec((bm, half), lambda i: (i, 0)),
          pl.BlockSpec((bm, half), lambda i: (i, 0)),
          pl.BlockSpec((bm, de2), lambda i: (i, 0)),
          pl.BlockSpec((bm, de2), lambda i: (i, 0)),
          pl.BlockSpec((d, d), lambda i: (0, 0)),
          pl.BlockSpec((d, de), lambda i: (0, 0)),
          pl.BlockSpec((d, 2 * d), lambda i: (0, 0)),
          pl.BlockSpec((1, d), lambda i: (0, 0)),
          pl.BlockSpec((1, d), lambda i: (0, 0)),
          pl.BlockSpec((1, d), lambda i: (0, 0)),
      ],
      out_specs=pl.BlockSpec((bm, d), lambda i: (i, 0)),
      out_shape=jax.ShapeDtypeStruct((n, d), jnp.float32),
  )(features, gl, gr, fea, feb, W_neigh, W_edge, W_lin,
    b_neigh.reshape(1, d), b_edge.reshape(1, d), b_lin.reshape(1, d))


def kernel(features, edge_index, edge_features, W_neigh, b_neigh, W_edge,
           b_edge, W_lin, b_lin):
  n, d = features.shape
  e = edge_index.shape[1]
  de = edge_features.shape[1]
  half = d // 2

  e_pad = ((e + _NTILE * _CHUNK - 1) // (_NTILE * _CHUNK)
           * (_NTILE * _CHUNK))
  chf = e_pad // (_NTILE * _CHUNK)
  chg = 2 * chf
  n_pad = ((n + 1 + 127) // 128) * 128

  pad = e_pad - e
  src = jnp.concatenate([edge_index[1], jnp.zeros((pad,), jnp.int32)])
  dst = jnp.concatenate([edge_index[0], jnp.full((pad,), n, jnp.int32)])
  src2 = jnp.concatenate([src, src + n])
  f2 = jnp.concatenate([features[:, :half], features[:, half:]], axis=0)
  ef2 = jnp.concatenate([
      edge_features,
      jnp.ones((e, 1), jnp.float32),
      jnp.zeros((e, 128 - de - 1), jnp.float32),
  ], axis=1)
  ef2 = jnp.pad(ef2, ((0, pad), (0, 0)))
  z_acc = jnp.zeros((n_pad, half), jnp.float32)
  z_fe = jnp.zeros((n_pad, 128), jnp.float32)

  g2 = _sc_gather_segsum(f2, src2, dst, z_acc, n_pad, half, chg, e_pad)
  fe2 = _sc_edge_segsum(ef2, dst, z_fe, n_pad, chf, e_pad)

  return _tc_epilogue(features, g2[0, :n], g2[1, :n], fe2[0, :n], fe2[1, :n],
                      W_neigh, W_edge, W_lin, b_neigh, b_edge, b_lin,
                      n, d, half, de)


# speedup vs baseline: 2.5402x; 1.1857x over previous
import functools

import jax
import jax.numpy as jnp
from jax import lax
from jax.experimental import pallas as pl
from jax.experimental.pallas import tpu as pltpu
from jax.experimental.pallas import tpu_sc as plsc

_CHUNK = 64
_GCHUNK = 128
_NSUB = 16
_NTILE = 32


def _sc_gather_segsum(f2, src2_2d, dstg, z_acc, n_pad, half, chg, e_pad):
  rows_per_tile = n_pad // _NSUB
  mesh = plsc.VectorSubcoreMesh(core_axis_name="c", subcore_axis_name="s")

  @functools.partial(
      pl.kernel,
      mesh=mesh,
      out_type=jax.ShapeDtypeStruct((2, n_pad, half), jnp.float32),
      scratch_types=[
          pltpu.VMEM_SHARED((n_pad, half), jnp.float32),
          pltpu.VMEM((chg, _GCHUNK), jnp.int32),
          pltpu.VMEM((_GCHUNK,), jnp.int32),
          pltpu.VMEM((_GCHUNK,), jnp.int32),
          pltpu.VMEM((_GCHUNK, half), jnp.float32),
          pltpu.VMEM((_GCHUNK, half), jnp.float32),
          pltpu.SemaphoreType.DMA,
          pltpu.SemaphoreType.DMA,
          pltpu.SemaphoreType.DMA,
          pltpu.SemaphoreType.DMA,
      ],
  )
  def k(f2_h, src2_h, dstg_h, zacc_h, g2_h, acc_sh, src_b, dst0, dst1,
        rows0, rows1, sem0, sem1, semd0, semd1):
    cid = lax.axis_index("c")
    sid = lax.axis_index("s")
    r0 = sid * rows_per_tile

    pltpu.sync_copy(zacc_h.at[pl.ds(r0, rows_per_tile)],
                    acc_sh.at[pl.ds(r0, rows_per_tile)])

    pltpu.sync_copy(src2_h.at[pl.ds(cid * (e_pad // _GCHUNK) + sid * chg,
                                    chg)], src_b)
    plsc.subcore_barrier()

    dbase = sid * (chg * _GCHUNK)

    def start_chunk(j, rows, dst, semr, semd):
      pltpu.make_async_copy(f2_h.at[src_b.at[j]], rows, semr).start()
      pltpu.make_async_copy(dstg_h.at[pl.ds(dbase + j * _GCHUNK, _GCHUNK)],
                            dst, semd).start()

    def finish_chunk(j, rows, dst, semr, semd):
      pltpu.make_async_copy(f2_h.at[src_b.at[j]], rows, semr).wait()
      pltpu.make_async_copy(dstg_h.at[pl.ds(dbase + j * _GCHUNK, _GCHUNK)],
                            dst, semd).wait()
      pltpu.sync_copy(rows, acc_sh.at[dst], add=True)

    start_chunk(0, rows0, dst0, sem0, semd0)

    def g_body2(t, carry):
      j0 = 2 * t
      j1 = j0 + 1
      start_chunk(j1, rows1, dst1, sem1, semd1)
      finish_chunk(j0, rows0, dst0, sem0, semd0)

      @pl.when(j1 + 1 < chg)
      def _():
        start_chunk(j1 + 1, rows0, dst0, sem0, semd0)

      finish_chunk(j1, rows1, dst1, sem1, semd1)
      return carry

    lax.fori_loop(0, chg // 2, g_body2, 0)
    plsc.subcore_barrier()

    pltpu.sync_copy(acc_sh.at[pl.ds(r0, rows_per_tile)],
                    g2_h.at[cid, pl.ds(r0, rows_per_tile)])

  return k(f2, src2_2d, dstg, z_acc)


def _sc_edge_segsum(ef2, dstg, z_fe, n_pad, chf, e_pad):
  rows_per_tile = n_pad // _NSUB
  mesh = plsc.VectorSubcoreMesh(core_axis_name="c", subcore_axis_name="s")

  @functools.partial(
      pl.kernel,
      mesh=mesh,
      out_type=jax.ShapeDtypeStruct((2, n_pad, 128), jnp.float32),
      scratch_types=[
          pltpu.VMEM_SHARED((n_pad, 128), jnp.float32),
          pltpu.VMEM((_CHUNK,), jnp.int32),
          pltpu.VMEM((_CHUNK, 128), jnp.float32),
      ],
  )
  def k(ef2_h, dstg_h, zfe_h, fe2_h, fe_sh, dst_v, ef_v):
    cid = lax.axis_index("c")
    sid = lax.axis_index("s")
    r0 = sid * rows_per_tile

    pltpu.sync_copy(zfe_h.at[pl.ds(r0, rows_per_tile)],
                    fe_sh.at[pl.ds(r0, rows_per_tile)])
    plsc.subcore_barrier()

    fbase = (cid * _NSUB + sid) * (chf * _CHUNK)

    def f_body(i, carry):
      off = fbase + i * _CHUNK
      pltpu.sync_copy(ef2_h.at[pl.ds(off, _CHUNK)], ef_v)
      pltpu.sync_copy(dstg_h.at[pl.ds(off, _CHUNK)], dst_v)
      pltpu.sync_copy(ef_v, fe_sh.at[dst_v], add=True)
      return carry

    lax.fori_loop(0, chf, f_body, 0)
    plsc.subcore_barrier()

    pltpu.sync_copy(fe_sh.at[pl.ds(r0, rows_per_tile)],
                    fe2_h.at[cid, pl.ds(r0, rows_per_tile)])

  return k(ef2, dstg, z_fe)


def _tc_epilogue(features, gl, gr, fea, feb, W_neigh, W_edge, W_lin,
                 b_neigh, b_edge, b_lin, n, d, half, de):
  bm = 1000
  grid = (n // bm,)

  def body(f_ref, gl_ref, gr_ref, fea_ref, feb_ref, wn_ref, we_ref, wl_ref,
           bn_ref, be_ref, bl_ref, out_ref):
    dot = lambda a, b: lax.dot_general(
        a, b, (((1,), (1,)), ((), ())), preferred_element_type=jnp.float32)
    f = f_ref[...]
    fe = fea_ref[...] + feb_ref[...]
    counts = fe[:, de:de + 1]
    hsum = dot(gl_ref[...], wn_ref[:, :half]) + dot(gr_ref[...], wn_ref[:, half:])
    sums = hsum + dot(fe[:, :de], we_ref[...])
    sums = sums + counts * (bn_ref[...] + be_ref[...]) + f
    s = sums / jnp.maximum(counts, 1.0)
    out_ref[...] = dot(f, wl_ref[:, :d]) + dot(s, wl_ref[:, d:]) + bl_ref[...]

  de2 = fea.shape[1]
  return pl.pallas_call(
      body,
      grid=grid,
      in_specs=[
          pl.BlockSpec((bm, d), lambda i: (i, 0)),
          pl.BlockSpec((bm, half), lambda i: (i, 0)),
          pl.BlockSpec((bm, half), lambda i: (i, 0)),
          pl.BlockSpec((bm, de2), lambda i: (i, 0)),
          pl.BlockSpec((bm, de2), lambda i: (i, 0)),
          pl.BlockSpec((d, d), lambda i: (0, 0)),
          pl.BlockSpec((d, de), lambda i: (0, 0)),
          pl.BlockSpec((d, 2 * d), lambda i: (0, 0)),
          pl.BlockSpec((1, d), lambda i: (0, 0)),
          pl.BlockSpec((1, d), lambda i: (0, 0)),
          pl.BlockSpec((1, d), lambda i: (0, 0)),
      ],
      out_specs=pl.BlockSpec((bm, d), lambda i: (i, 0)),
      out_shape=jax.ShapeDtypeStruct((n, d), jnp.float32),
  )(features, gl, gr, fea, feb, W_neigh, W_edge, W_lin,
    b_neigh.reshape(1, d), b_edge.reshape(1, d), b_lin.reshape(1, d))


def kernel(features, edge_index, edge_features, W_neigh, b_neigh, W_edge,
           b_edge, W_lin, b_lin):
  n, d = features.shape
  e = edge_index.shape[1]
  de = edge_features.shape[1]
  half = d // 2

  e_pad = ((e + 2 * _NSUB * _GCHUNK - 1) // (2 * _NSUB * _GCHUNK)
           * (2 * _NSUB * _GCHUNK))
  chg = e_pad // (_NSUB * _GCHUNK)
  chf = e_pad // (_NTILE * _CHUNK)
  n_pad = ((n + 1 + 127) // 128) * 128

  pad = e_pad - e
  src = jnp.concatenate([edge_index[1], jnp.zeros((pad,), jnp.int32)])
  dst = jnp.concatenate([edge_index[0], jnp.full((pad,), n, jnp.int32)])
  src2 = jnp.concatenate([src, src + n])
  src2_2d = src2.reshape(2 * e_pad // _GCHUNK, _GCHUNK)
  f2 = jnp.concatenate([features[:, :half], features[:, half:]], axis=0)
  ef2 = jnp.concatenate([
      edge_features,
      jnp.ones((e, 1), jnp.float32),
      jnp.zeros((e, 128 - de - 1), jnp.float32),
  ], axis=1)
  ef2 = jnp.pad(ef2, ((0, pad), (0, 0)))
  z_acc = jnp.zeros((n_pad, half), jnp.float32)
  z_fe = jnp.zeros((n_pad, 128), jnp.float32)

  g2 = _sc_gather_segsum(f2, src2_2d, dst, z_acc, n_pad, half, chg, e_pad)
  fe2 = _sc_edge_segsum(ef2, dst, z_fe, n_pad, chf, e_pad)

  return _tc_epilogue(features, g2[0, :n], g2[1, :n], fe2[0, :n], fe2[1, :n],
                      W_neigh, W_edge, W_lin, b_neigh, b_edge, b_lin,
                      n, d, half, de)


# speedup vs baseline: 2.9910x; 1.1775x over previous
import functools

import jax
import jax.numpy as jnp
from jax import lax
from jax.experimental import pallas as pl
from jax.experimental.pallas import tpu as pltpu
from jax.experimental.pallas import tpu_sc as plsc

_GCHUNK = 128
_NSUB = 16
_NTILE = 32


def _sc_gather_segsum(f2, src2_2d, dstg, z_acc, n_pad, half, chg, e_pad):
  rows_per_tile = n_pad // _NSUB
  mesh = plsc.VectorSubcoreMesh(core_axis_name="c", subcore_axis_name="s")

  @functools.partial(
      pl.kernel,
      mesh=mesh,
      out_type=jax.ShapeDtypeStruct((2, n_pad, half), jnp.float32),
      scratch_types=[
          pltpu.VMEM_SHARED((n_pad, half), jnp.float32),
          pltpu.VMEM((chg, _GCHUNK), jnp.int32),
          pltpu.VMEM((_GCHUNK,), jnp.int32),
          pltpu.VMEM((_GCHUNK,), jnp.int32),
          pltpu.VMEM((_GCHUNK, half), jnp.float32),
          pltpu.VMEM((_GCHUNK, half), jnp.float32),
          pltpu.SemaphoreType.DMA,
          pltpu.SemaphoreType.DMA,
          pltpu.SemaphoreType.DMA,
          pltpu.SemaphoreType.DMA,
      ],
  )
  def k(f2_h, src2_h, dstg_h, zacc_h, g2_h, acc_sh, src_b, dst0, dst1,
        rows0, rows1, sem0, sem1, semd0, semd1):
    cid = lax.axis_index("c")
    sid = lax.axis_index("s")
    r0 = sid * rows_per_tile

    pltpu.sync_copy(zacc_h.at[pl.ds(r0, rows_per_tile)],
                    acc_sh.at[pl.ds(r0, rows_per_tile)])

    pltpu.sync_copy(src2_h.at[pl.ds(cid * (e_pad // _GCHUNK) + sid * chg,
                                    chg)], src_b)
    plsc.subcore_barrier()

    dbase = sid * (chg * _GCHUNK)

    def start_chunk(j, rows, dst, semr, semd):
      pltpu.make_async_copy(f2_h.at[src_b.at[j]], rows, semr).start()
      pltpu.make_async_copy(dstg_h.at[pl.ds(dbase + j * _GCHUNK, _GCHUNK)],
                            dst, semd).start()

    def finish_chunk(j, rows, dst, semr, semd):
      pltpu.make_async_copy(f2_h.at[src_b.at[j]], rows, semr).wait()
      pltpu.make_async_copy(dstg_h.at[pl.ds(dbase + j * _GCHUNK, _GCHUNK)],
                            dst, semd).wait()
      pltpu.sync_copy(rows, acc_sh.at[dst], add=True)

    start_chunk(0, rows0, dst0, sem0, semd0)

    def g_body2(t, carry):
      j0 = 2 * t
      j1 = j0 + 1
      start_chunk(j1, rows1, dst1, sem1, semd1)
      finish_chunk(j0, rows0, dst0, sem0, semd0)

      @pl.when(j1 + 1 < chg)
      def _():
        start_chunk(j1 + 1, rows0, dst0, sem0, semd0)

      finish_chunk(j1, rows1, dst1, sem1, semd1)
      return carry

    lax.fori_loop(0, chg // 2, g_body2, 0)
    plsc.subcore_barrier()

    pltpu.sync_copy(acc_sh.at[pl.ds(r0, rows_per_tile)],
                    g2_h.at[cid, pl.ds(r0, rows_per_tile)])

  return k(f2, src2_2d, dstg, z_acc)


def _sc_edge_segsum(ef2, dstg, z_fe, n_pad, chf, e_pad):
  rows_per_tile = n_pad // _NSUB
  mesh = plsc.VectorSubcoreMesh(core_axis_name="c", subcore_axis_name="s")

  @functools.partial(
      pl.kernel,
      mesh=mesh,
      out_type=jax.ShapeDtypeStruct((2, n_pad, 128), jnp.float32),
      scratch_types=[
          pltpu.VMEM_SHARED((n_pad, 128), jnp.float32),
          pltpu.VMEM((_GCHUNK,), jnp.int32),
          pltpu.VMEM((_GCHUNK,), jnp.int32),
          pltpu.VMEM((_GCHUNK, 128), jnp.float32),
          pltpu.VMEM((_GCHUNK, 128), jnp.float32),
          pltpu.SemaphoreType.DMA,
          pltpu.SemaphoreType.DMA,
          pltpu.SemaphoreType.DMA,
          pltpu.SemaphoreType.DMA,
      ],
  )
  def k(ef2_h, dstg_h, zfe_h, fe2_h, fe_sh, dst0, dst1, ef0, ef1,
        sem0, sem1, semd0, semd1):
    cid = lax.axis_index("c")
    sid = lax.axis_index("s")
    r0 = sid * rows_per_tile

    pltpu.sync_copy(zfe_h.at[pl.ds(r0, rows_per_tile)],
                    fe_sh.at[pl.ds(r0, rows_per_tile)])
    plsc.subcore_barrier()

    fbase = (cid * _NSUB + sid) * (chf * _GCHUNK)

    def start_chunk(j, ef, dst, semr, semd):
      off = fbase + j * _GCHUNK
      pltpu.make_async_copy(ef2_h.at[pl.ds(off, _GCHUNK)], ef, semr).start()
      pltpu.make_async_copy(dstg_h.at[pl.ds(off, _GCHUNK)], dst, semd).start()

    def finish_chunk(j, ef, dst, semr, semd):
      off = fbase + j * _GCHUNK
      pltpu.make_async_copy(ef2_h.at[pl.ds(off, _GCHUNK)], ef, semr).wait()
      pltpu.make_async_copy(dstg_h.at[pl.ds(off, _GCHUNK)], dst, semd).wait()
      pltpu.sync_copy(ef, fe_sh.at[dst], add=True)

    start_chunk(0, ef0, dst0, sem0, semd0)

    def f_body2(t, carry):
      j0 = 2 * t
      j1 = j0 + 1
      start_chunk(j1, ef1, dst1, sem1, semd1)
      finish_chunk(j0, ef0, dst0, sem0, semd0)

      @pl.when(j1 + 1 < chf)
      def _():
        start_chunk(j1 + 1, ef0, dst0, sem0, semd0)

      finish_chunk(j1, ef1, dst1, sem1, semd1)
      return carry

    lax.fori_loop(0, chf // 2, f_body2, 0)
    plsc.subcore_barrier()

    pltpu.sync_copy(fe_sh.at[pl.ds(r0, rows_per_tile)],
                    fe2_h.at[cid, pl.ds(r0, rows_per_tile)])

  return k(ef2, dstg, z_fe)


def _tc_epilogue(features, gl, gr, fea, feb, W_neigh, W_edge, W_lin,
                 b_neigh, b_edge, b_lin, n, d, half, de):
  bm = 1000
  grid = (n // bm,)

  def body(f_ref, gl_ref, gr_ref, fea_ref, feb_ref, wn_ref, we_ref, wl_ref,
           bn_ref, be_ref, bl_ref, out_ref):
    dot = lambda a, b: lax.dot_general(
        a, b, (((1,), (1,)), ((), ())), preferred_element_type=jnp.float32)
    f = f_ref[...]
    fe = fea_ref[...] + feb_ref[...]
    counts = fe[:, de:de + 1]
    hsum = dot(gl_ref[...], wn_ref[:, :half]) + dot(gr_ref[...], wn_ref[:, half:])
    sums = hsum + dot(fe[:, :de], we_ref[...])
    sums = sums + counts * (bn_ref[...] + be_ref[...]) + f
    s = sums / jnp.maximum(counts, 1.0)
    out_ref[...] = dot(f, wl_ref[:, :d]) + dot(s, wl_ref[:, d:]) + bl_ref[...]

  de2 = fea.shape[1]
  return pl.pallas_call(
      body,
      grid=grid,
      in_specs=[
          pl.BlockSpec((bm, d), lambda i: (i, 0)),
          pl.BlockSpec((bm, half), lambda i: (i, 0)),
          pl.BlockSpec((bm, half), lambda i: (i, 0)),
          pl.BlockSpec((bm, de2), lambda i: (i, 0)),
          pl.BlockSpec((bm, de2), lambda i: (i, 0)),
          pl.BlockSpec((d, d), lambda i: (0, 0)),
          pl.BlockSpec((d, de), lambda i: (0, 0)),
          pl.BlockSpec((d, 2 * d), lambda i: (0, 0)),
          pl.BlockSpec((1, d), lambda i: (0, 0)),
          pl.BlockSpec((1, d), lambda i: (0, 0)),
          pl.BlockSpec((1, d), lambda i: (0, 0)),
      ],
      out_specs=pl.BlockSpec((bm, d), lambda i: (i, 0)),
      out_shape=jax.ShapeDtypeStruct((n, d), jnp.float32),
  )(features, gl, gr, fea, feb, W_neigh, W_edge, W_lin,
    b_neigh.reshape(1, d), b_edge.reshape(1, d), b_lin.reshape(1, d))


def kernel(features, edge_index, edge_features, W_neigh, b_neigh, W_edge,
           b_edge, W_lin, b_lin):
  n, d = features.shape
  e = edge_index.shape[1]
  de = edge_features.shape[1]
  half = d // 2

  e_pad = ((e + 2 * _NSUB * _GCHUNK - 1) // (2 * _NSUB * _GCHUNK)
           * (2 * _NSUB * _GCHUNK))
  chg = e_pad // (_NSUB * _GCHUNK)
  chf = e_pad // (_NTILE * _GCHUNK)
  n_pad = ((n + 1 + 127) // 128) * 128

  pad = e_pad - e
  src = jnp.concatenate([edge_index[1], jnp.zeros((pad,), jnp.int32)])
  dst = jnp.concatenate([edge_index[0], jnp.full((pad,), n, jnp.int32)])
  src2 = jnp.concatenate([src, src + n])
  src2_2d = src2.reshape(2 * e_pad // _GCHUNK, _GCHUNK)
  f2 = jnp.concatenate([features[:, :half], features[:, half:]], axis=0)
  ef2 = jnp.concatenate([
      edge_features,
      jnp.ones((e, 1), jnp.float32),
      jnp.zeros((e, 128 - de - 1), jnp.float32),
  ], axis=1)
  ef2 = jnp.pad(ef2, ((0, pad), (0, 0)))
  z_acc = jnp.zeros((n_pad, half), jnp.float32)
  z_fe = jnp.zeros((n_pad, 128), jnp.float32)

  g2 = _sc_gather_segsum(f2, src2_2d, dst, z_acc, n_pad, half, chg, e_pad)
  fe2 = _sc_edge_segsum(ef2, dst, z_fe, n_pad, chf, e_pad)

  return _tc_epilogue(features, g2[0, :n], g2[1, :n], fe2[0, :n], fe2[1, :n],
                      W_neigh, W_edge, W_lin, b_neigh, b_edge, b_lin,
                      n, d, half, de)


# speedup vs baseline: 3.0881x; 1.0325x over previous
import functools

import jax
import jax.numpy as jnp
from jax import lax
from jax.experimental import pallas as pl
from jax.experimental.pallas import tpu as pltpu
from jax.experimental.pallas import tpu_sc as plsc

_GCHUNK = 128
_SCHUNK = 64
_NSUB = 16
_NTILE = 32


def _sc_gather_segsum(f2, src2, dstg, z_acc, n_pad, half, chg, e_pad):
  rows_per_tile = n_pad // _NSUB
  mesh = plsc.VectorSubcoreMesh(core_axis_name="c", subcore_axis_name="s")

  @functools.partial(
      pl.kernel,
      mesh=mesh,
      out_type=jax.ShapeDtypeStruct((2, n_pad, half), jnp.float32),
      scratch_types=[
          pltpu.VMEM_SHARED((n_pad, half), jnp.float32),
          pltpu.VMEM((_SCHUNK,), jnp.int32),
          pltpu.VMEM((_SCHUNK,), jnp.int32),
          pltpu.VMEM((_SCHUNK,), jnp.int32),
          pltpu.VMEM((_SCHUNK,), jnp.int32),
          pltpu.VMEM((_SCHUNK,), jnp.int32),
          pltpu.VMEM((_SCHUNK,), jnp.int32),
          pltpu.VMEM((_SCHUNK,), jnp.int32),
          pltpu.VMEM((_SCHUNK,), jnp.int32),
          pltpu.VMEM((_SCHUNK, half), jnp.float32),
          pltpu.VMEM((_SCHUNK, half), jnp.float32),
          pltpu.VMEM((_SCHUNK, half), jnp.float32),
          pltpu.VMEM((_SCHUNK, half), jnp.float32),
          pltpu.SemaphoreType.DMA,
          pltpu.SemaphoreType.DMA,
          pltpu.SemaphoreType.DMA,
          pltpu.SemaphoreType.DMA,
          pltpu.SemaphoreType.DMA,
          pltpu.SemaphoreType.DMA,
          pltpu.SemaphoreType.DMA,
          pltpu.SemaphoreType.DMA,
      ],
  )
  def k(f2_h, src2_h, dstg_h, zacc_h, g2_h, acc_sh,
        src0, src1, src2_, src3, dst0, dst1, dst2, dst3,
        rows0, rows1, rows2, rows3,
        semi0, semi1, semi2, semi3, semg0, semg1, semg2, semg3):
    cid = lax.axis_index("c")
    sid = lax.axis_index("s")
    r0 = sid * rows_per_tile

    pltpu.sync_copy(zacc_h.at[pl.ds(r0, rows_per_tile)],
                    acc_sh.at[pl.ds(r0, rows_per_tile)])
    plsc.subcore_barrier()

    dbase = sid * (chg * _SCHUNK)
    sbase = cid * e_pad + dbase
    sets = ((src0, dst0, rows0, semi0, semg0),
            (src1, dst1, rows1, semi1, semg1),
            (src2_, dst2, rows2, semi2, semg2),
            (src3, dst3, rows3, semi3, semg3))

    def start_idx(j, src, dst, rows, semi, semg):
      del rows, semg
      pltpu.make_async_copy(src2_h.at[pl.ds(sbase + j * _SCHUNK, _SCHUNK)],
                            src, semi).start()
      pltpu.make_async_copy(dstg_h.at[pl.ds(dbase + j * _SCHUNK, _SCHUNK)],
                            dst, semi).start()

    def start_gather(j, src, dst, rows, semi, semg):
      pltpu.make_async_copy(src2_h.at[pl.ds(sbase + j * _SCHUNK, _SCHUNK)],
                            src, semi).wait()
      pltpu.make_async_copy(dstg_h.at[pl.ds(dbase + j * _SCHUNK, _SCHUNK)],
                            dst, semi).wait()
      pltpu.make_async_copy(f2_h.at[src], rows, semg).start()

    def finish_chunk(j, src, dst, rows, semi, semg):
      del semi
      pltpu.make_async_copy(f2_h.at[src], rows, semg).wait()
      pltpu.sync_copy(rows, acc_sh.at[dst], add=True)

    start_idx(0, *sets[0])
    start_idx(1, *sets[1])
    start_idx(2, *sets[2])
    start_gather(0, *sets[0])
    start_gather(1, *sets[1])

    def g_body4(t, carry):
      j0 = 4 * t
      for s in range(4):
        j = j0 + s
        @pl.when(j + 3 < chg)
        def _(j=j, s=s):
          start_idx(j + 3, *sets[(s + 3) % 4])

        @pl.when(j + 2 < chg)
        def _(j=j, s=s):
          start_gather(j + 2, *sets[(s + 2) % 4])

        finish_chunk(j, *sets[s])
      return carry

    lax.fori_loop(0, chg // 4, g_body4, 0)
    plsc.subcore_barrier()

    pltpu.sync_copy(acc_sh.at[pl.ds(r0, rows_per_tile)],
                    g2_h.at[cid, pl.ds(r0, rows_per_tile)])

  return k(f2, src2, dstg, z_acc)


def _sc_edge_segsum(ef2, dstg, z_fe, n_pad, chf, e_pad):
  rows_per_tile = n_pad // _NSUB
  mesh = plsc.VectorSubcoreMesh(core_axis_name="c", subcore_axis_name="s")

  @functools.partial(
      pl.kernel,
      mesh=mesh,
      out_type=jax.ShapeDtypeStruct((2, n_pad, 128), jnp.float32),
      scratch_types=[
          pltpu.VMEM_SHARED((n_pad, 128), jnp.float32),
          pltpu.VMEM((_GCHUNK,), jnp.int32),
          pltpu.VMEM((_GCHUNK,), jnp.int32),
          pltpu.VMEM((_GCHUNK, 128), jnp.float32),
          pltpu.VMEM((_GCHUNK, 128), jnp.float32),
          pltpu.SemaphoreType.DMA,
          pltpu.SemaphoreType.DMA,
          pltpu.SemaphoreType.DMA,
          pltpu.SemaphoreType.DMA,
      ],
  )
  def k(ef2_h, dstg_h, zfe_h, fe2_h, fe_sh, dst0, dst1, ef0, ef1,
        sem0, sem1, semd0, semd1):
    cid = lax.axis_index("c")
    sid = lax.axis_index("s")
    r0 = sid * rows_per_tile

    pltpu.sync_copy(zfe_h.at[pl.ds(r0, rows_per_tile)],
                    fe_sh.at[pl.ds(r0, rows_per_tile)])
    plsc.subcore_barrier()

    fbase = (cid * _NSUB + sid) * (chf * _GCHUNK)

    def start_chunk(j, ef, dst, semr, semd):
      off = fbase + j * _GCHUNK
      pltpu.make_async_copy(ef2_h.at[pl.ds(off, _GCHUNK)], ef, semr).start()
      pltpu.make_async_copy(dstg_h.at[pl.ds(off, _GCHUNK)], dst, semd).start()

    def finish_chunk(j, ef, dst, semr, semd):
      off = fbase + j * _GCHUNK
      pltpu.make_async_copy(ef2_h.at[pl.ds(off, _GCHUNK)], ef, semr).wait()
      pltpu.make_async_copy(dstg_h.at[pl.ds(off, _GCHUNK)], dst, semd).wait()
      pltpu.sync_copy(ef, fe_sh.at[dst], add=True)

    start_chunk(0, ef0, dst0, sem0, semd0)

    def f_body2(t, carry):
      j0 = 2 * t
      j1 = j0 + 1
      start_chunk(j1, ef1, dst1, sem1, semd1)
      finish_chunk(j0, ef0, dst0, sem0, semd0)

      @pl.when(j1 + 1 < chf)
      def _():
        start_chunk(j1 + 1, ef0, dst0, sem0, semd0)

      finish_chunk(j1, ef1, dst1, sem1, semd1)
      return carry

    lax.fori_loop(0, chf // 2, f_body2, 0)
    plsc.subcore_barrier()

    pltpu.sync_copy(fe_sh.at[pl.ds(r0, rows_per_tile)],
                    fe2_h.at[cid, pl.ds(r0, rows_per_tile)])

  return k(ef2, dstg, z_fe)


def _tc_epilogue(features, gl, gr, fea, feb, W_neigh, W_edge, W_lin,
                 b_neigh, b_edge, b_lin, n, d, half, de):
  bm = 1000
  grid = (n // bm,)

  def body(f_ref, gl_ref, gr_ref, fea_ref, feb_ref, wn_ref, we_ref, wl_ref,
           bn_ref, be_ref, bl_ref, out_ref):
    dot = lambda a, b: lax.dot_general(
        a, b, (((1,), (1,)), ((), ())), preferred_element_type=jnp.float32)
    f = f_ref[...]
    fe = fea_ref[...] + feb_ref[...]
    counts = fe[:, de:de + 1]
    hsum = dot(gl_ref[...], wn_ref[:, :half]) + dot(gr_ref[...], wn_ref[:, half:])
    sums = hsum + dot(fe[:, :de], we_ref[...])
    sums = sums + counts * (bn_ref[...] + be_ref[...]) + f
    s = sums / jnp.maximum(counts, 1.0)
    out_ref[...] = dot(f, wl_ref[:, :d]) + dot(s, wl_ref[:, d:]) + bl_ref[...]

  de2 = fea.shape[1]
  return pl.pallas_call(
      body,
      grid=grid,
      in_specs=[
          pl.BlockSpec((bm, d), lambda i: (i, 0)),
          pl.BlockSpec((bm, half), lambda i: (i, 0)),
          pl.BlockSpec((bm, half), lambda i: (i, 0)),
          pl.BlockSpec((bm, de2), lambda i: (i, 0)),
          pl.BlockSpec((bm, de2), lambda i: (i, 0)),
          pl.BlockSpec((d, d), lambda i: (0, 0)),
          pl.BlockSpec((d, de), lambda i: (0, 0)),
          pl.BlockSpec((d, 2 * d), lambda i: (0, 0)),
          pl.BlockSpec((1, d), lambda i: (0, 0)),
          pl.BlockSpec((1, d), lambda i: (0, 0)),
          pl.BlockSpec((1, d), lambda i: (0, 0)),
      ],
      out_specs=pl.BlockSpec((bm, d), lambda i: (i, 0)),
      out_shape=jax.ShapeDtypeStruct((n, d), jnp.float32),
  )(features, gl, gr, fea, feb, W_neigh, W_edge, W_lin,
    b_neigh.reshape(1, d), b_edge.reshape(1, d), b_lin.reshape(1, d))


def kernel(features, edge_index, edge_features, W_neigh, b_neigh, W_edge,
           b_edge, W_lin, b_lin):
  n, d = features.shape
  e = edge_index.shape[1]
  de = edge_features.shape[1]
  half = d // 2

  e_pad = ((e + 2 * _NSUB * _GCHUNK - 1) // (2 * _NSUB * _GCHUNK)
           * (2 * _NSUB * _GCHUNK))
  chg = e_pad // (_NSUB * _SCHUNK)
  chf = e_pad // (_NTILE * _GCHUNK)
  n_pad = ((n + 1 + 127) // 128) * 128

  pad = e_pad - e
  src = jnp.concatenate([edge_index[1], jnp.zeros((pad,), jnp.int32)])
  dst = jnp.concatenate([edge_index[0], jnp.full((pad,), n, jnp.int32)])
  src2 = jnp.concatenate([src, src + n])
  f2 = jnp.concatenate([features[:, :half], features[:, half:]], axis=0)
  ef2 = jnp.concatenate([
      edge_features,
      jnp.ones((e, 1), jnp.float32),
      jnp.zeros((e, 128 - de - 1), jnp.float32),
  ], axis=1)
  ef2 = jnp.pad(ef2, ((0, pad), (0, 0)))
  z_acc = jnp.zeros((n_pad, half), jnp.float32)
  z_fe = jnp.zeros((n_pad, 128), jnp.float32)

  g2 = _sc_gather_segsum(f2, src2, dst, z_acc, n_pad, half, chg, e_pad)
  fe2 = _sc_edge_segsum(ef2, dst, z_fe, n_pad, chf, e_pad)

  return _tc_epilogue(features, g2[0, :n], g2[1, :n], fe2[0, :n], fe2[1, :n],
                      W_neigh, W_edge, W_lin, b_neigh, b_edge, b_lin,
                      n, d, half, de)
